# baseline (speedup 1.0000x reference)
import functools

import jax
import jax.numpy as jnp
from jax import lax
from jax.experimental import pallas as pl
from jax.experimental.pallas import tpu as pltpu
from jax.experimental.pallas import tpu_sc as plsc

MAX_POS = 77
DIM = 768
HALF = DIM // 2
BATCH = 1024
SEQ = 77
CH = 16
FPB = 64 // CH
TAIL = SEQ - FPB * CH
NT = 16
BPT = BATCH // NT
FCP = BPT * FPB
LANES = 16


def _make_kernel():
  mesh = plsc.VectorSubcoreMesh(core_axis_name="c", subcore_axis_name="s")

  @functools.partial(
      pl.kernel,
      mesh=mesh,
      out_type=jax.ShapeDtypeStruct((BATCH, SEQ, DIM), jnp.float32),
      scratch_types=[
          pltpu.VMEM((FCP, CH), jnp.int32),
          pltpu.VMEM((FCP, CH), jnp.int32),
          pltpu.VMEM((BPT, LANES), jnp.int32),
          pltpu.VMEM((BPT, LANES), jnp.int32),
          pltpu.VMEM((MAX_POS, HALF), jnp.float32),
          pltpu.VMEM((CH, HALF), jnp.float32),
          pltpu.VMEM((CH, HALF), jnp.float32),
          pltpu.VMEM((TAIL, HALF), jnp.float32),
      ] + [pltpu.SemaphoreType.DMA for _ in range(5)],
  )
  def emb_kernel(idsf_hbm, pidsf_hbm, idst_hbm, pidst_hbm, tok_hbm, pos_hbm,
                 out_hbm, idxf, pidxf, idxt, pidxt, ptab,
                 f0, f1, t0, gf0, gf1, sf0, sf1, st0):
    fbufs = [f0, f1]
    fgsems = [gf0, gf1]
    fssems = [sf0, sf1]
    tile = lax.axis_index("s")
    c0 = lax.axis_index("c") * HALF

    def compute(dst, srcbuf, pvec, nrows):
      for i in range(nrows):
        p = pvec[i]

        @plsc.parallel_loop(0, HALF // LANES, step=12)
        def _(j, i=i, p=p, dst=dst, srcbuf=srcbuf):
          sls = [pl.ds((j + j8) * LANES, LANES) for j8 in range(12)]
          vals = [srcbuf[i, sl] + ptab[p, sl] for sl in sls]
          for j8 in range(12):
            dst[i, sls[j8]] = vals[j8]

    def f_gather(g, b):
      pltpu.async_copy(tok_hbm.at[idxf.at[g], pl.ds(c0, HALF)],
                       fbufs[b], fgsems[b])

    def f_wait_gather(b):
      pltpu.make_async_copy(tok_hbm.at[idxf.at[0], pl.ds(c0, HALF)],
                            fbufs[b], fgsems[b]).wait()

    def f_store_dst(g):
      k = g // FPB
      m = g - k * FPB
      return out_hbm.at[tile * BPT + k, pl.ds(m * CH, CH), pl.ds(c0, HALF)]

    def f_store(g, b):
      pltpu.async_copy(fbufs[b], f_store_dst(g), fssems[b])

    def f_wait_store(g, b):
      pltpu.make_async_copy(fbufs[b], f_store_dst(g), fssems[b]).wait()

    def t_gather(k, b):
      pltpu.async_copy(tok_hbm.at[idxt.at[k], pl.ds(c0, HALF)],
                       fbufs[b], fgsems[b])

    def t_store_dst(k):
      return out_hbm.at[tile * BPT + k, pl.ds(FPB * CH, TAIL),
                        pl.ds(c0, HALF)]

    def t_store(k):
      pltpu.async_copy(t0, t_store_dst(k), st0)

    def t_wait_store(k):
      pltpu.make_async_copy(t0, t_store_dst(k), st0).wait()

    pltpu.sync_copy(pos_hbm.at[:, pl.ds(c0, HALF)], ptab)
    pltpu.sync_copy(idsf_hbm.at[tile], idxf)
    pltpu.sync_copy(pidsf_hbm.at[tile], pidxf)
    pltpu.sync_copy(idst_hbm.at[tile], idxt)
    pltpu.sync_copy(pidst_hbm.at[tile], pidxt)

    f_gather(0, 0)
    f_gather(1, 1)

    def f_ring(i, carry):
      g = 2 * i
      f_wait_gather(0)
      compute(fbufs[0], fbufs[0], pidxf[g], CH)
      f_store(g, 0)
      f_wait_gather(1)
      compute(fbufs[1], fbufs[1], pidxf[g + 1], CH)
      f_store(g + 1, 1)

      @pl.when(i < FCP // 2 - 1)
      def _():
        f_wait_store(g, 0)
        f_gather(g + 2, 0)
        f_wait_store(g + 1, 1)
        f_gather(g + 3, 1)

      return carry

    lax.fori_loop(0, FCP // 2, f_ring, 0)

    f_wait_store(FCP - 2, 0)
    f_wait_store(FCP - 1, 1)

    t_gather(0, 0)
    t_gather(1, 1)

    def t_ring(kk, carry):
      k = 2 * kk
      f_wait_gather(0)

      @pl.when(kk > 0)
      def _():
        t_wait_store(k - 1)

      compute(t0, f0, pidxt[k], TAIL)
      t_store(k)

      @pl.when(k + 2 < BPT)
      def _():
        t_gather(k + 2, 0)

      f_wait_gather(1)
      t_wait_store(k)
      compute(t0, f1, pidxt[k + 1], TAIL)
      t_store(k + 1)

      @pl.when(k + 3 < BPT)
      def _():
        t_gather(k + 3, 1)

      return carry

    lax.fori_loop(0, BPT // 2, t_ring, 0)

    t_wait_store(BPT - 1)

  return emb_kernel


_emb_kernel = _make_kernel()


@jax.jit
def kernel(input_ids, position_ids, token_table, position_table):
  ids = input_ids.astype(jnp.int32)
  pids = position_ids.astype(jnp.int32)
  idsf = ids[:, :FPB * CH].reshape(NT, FCP, CH)
  pidsf = pids[:, :FPB * CH].reshape(NT, FCP, CH)
  idst = jnp.pad(ids[:, FPB * CH:],
                 ((0, 0), (0, LANES - TAIL))).reshape(NT, BPT, LANES)
  pidst = jnp.pad(pids[:, FPB * CH:],
                  ((0, 0), (0, LANES - TAIL))).reshape(NT, BPT, LANES)
  return _emb_kernel(idsf, pidsf, idst, pidst, token_table, position_table)


# speedup vs baseline: 1.4728x; 1.4728x over previous
import functools

import jax
import jax.numpy as jnp
from jax import lax
from jax.experimental import pallas as pl
from jax.experimental.pallas import tpu as pltpu
from jax.experimental.pallas import tpu_sc as plsc

MAX_POS = 77
DIM = 768
HALF = DIM // 2
BATCH = 1024
SEQ = 77
CH = 16
FPB = 64 // CH
TAIL = SEQ - FPB * CH
NT = 16
BPT = BATCH // NT
FCP = BPT * FPB
LANES = 16


def _make_kernel():
  mesh = plsc.VectorSubcoreMesh(core_axis_name="c", subcore_axis_name="s")

  @functools.partial(
      pl.kernel,
      mesh=mesh,
      out_type=jax.ShapeDtypeStruct((BATCH, SEQ, DIM), jnp.float32),
      scratch_types=[
          pltpu.VMEM((FCP, CH), jnp.int32),
          pltpu.VMEM((FCP, CH), jnp.int32),
          pltpu.VMEM((BPT, LANES), jnp.int32),
          pltpu.VMEM((BPT, LANES), jnp.int32),
          pltpu.VMEM((MAX_POS, HALF), jnp.float32),
          pltpu.VMEM((CH, HALF), jnp.float32),
          pltpu.VMEM((CH, HALF), jnp.float32),
          pltpu.VMEM((TAIL, HALF), jnp.float32),
      ] + [pltpu.SemaphoreType.DMA for _ in range(5)],
  )
  def emb_kernel(idsf_hbm, pidsf_hbm, idst_hbm, pidst_hbm, tok_hbm, pos_hbm,
                 out_hbm, idxf, pidxf, idxt, pidxt, ptab,
                 f0, f1, t0, gf0, gf1, sf0, sf1, st0):
    fbufs = [f0, f1]
    fgsems = [gf0, gf1]
    fssems = [sf0, sf1]
    tile = lax.axis_index("s")
    c0 = lax.axis_index("c") * HALF

    def compute(dst, srcbuf, pvec, nrows):
      for i in range(nrows):
        p = pvec[i]

        @plsc.parallel_loop(0, HALF // LANES, step=4)
        def _(j, i=i, p=p, dst=dst, srcbuf=srcbuf):
          sls = [pl.ds((j + j8) * LANES, LANES) for j8 in range(4)]
          vals = [srcbuf[i, sl] + ptab[p, sl] for sl in sls]
          for j8 in range(4):
            dst[i, sls[j8]] = vals[j8]

    def f_gather(g, b):
      pltpu.async_copy(tok_hbm.at[idxf.at[g], pl.ds(c0, HALF)],
                       fbufs[b], fgsems[b])

    def f_wait_gather(b):
      pltpu.make_async_copy(tok_hbm.at[idxf.at[0], pl.ds(c0, HALF)],
                            fbufs[b], fgsems[b]).wait()

    def f_store_dst(g):
      k = g // FPB
      m = g - k * FPB
      return out_hbm.at[tile * BPT + k, pl.ds(m * CH, CH), pl.ds(c0, HALF)]

    def f_store(g, b):
      pltpu.async_copy(fbufs[b], f_store_dst(g), fssems[b])

    def f_wait_store(g, b):
      pltpu.make_async_copy(fbufs[b], f_store_dst(g), fssems[b]).wait()

    def t_gather(k, b):
      pltpu.async_copy(tok_hbm.at[idxt.at[k], pl.ds(c0, HALF)],
                       fbufs[b], fgsems[b])

    def t_store_dst(k):
      return out_hbm.at[tile * BPT + k, pl.ds(FPB * CH, TAIL),
                        pl.ds(c0, HALF)]

    def t_store(k):
      pltpu.async_copy(t0, t_store_dst(k), st0)

    def t_wait_store(k):
      pltpu.make_async_copy(t0, t_store_dst(k), st0).wait()

    pltpu.sync_copy(pos_hbm.at[:, pl.ds(c0, HALF)], ptab)
    pltpu.sync_copy(idsf_hbm.at[tile], idxf)
    pltpu.sync_copy(pidsf_hbm.at[tile], pidxf)
    pltpu.sync_copy(idst_hbm.at[tile], idxt)
    pltpu.sync_copy(pidst_hbm.at[tile], pidxt)

    f_gather(0, 0)
    f_gather(1, 1)

    def f_ring(i, carry):
      g = 2 * i
      f_wait_gather(0)
      compute(fbufs[0], fbufs[0], pidxf[g], CH)
      f_store(g, 0)
      f_wait_gather(1)
      compute(fbufs[1], fbufs[1], pidxf[g + 1], CH)
      f_store(g + 1, 1)

      @pl.when(i < FCP // 2 - 1)
      def _():
        f_wait_store(g, 0)
        f_gather(g + 2, 0)
        f_wait_store(g + 1, 1)
        f_gather(g + 3, 1)

      return carry

    lax.fori_loop(0, FCP // 2, f_ring, 0)

    f_wait_store(FCP - 2, 0)
    f_wait_store(FCP - 1, 1)

    t_gather(0, 0)
    t_gather(1, 1)

    def t_ring(kk, carry):
      k = 2 * kk
      f_wait_gather(0)

      @pl.when(kk > 0)
      def _():
        t_wait_store(k - 1)

      compute(t0, f0, pidxt[k], TAIL)
      t_store(k)

      @pl.when(k + 2 < BPT)
      def _():
        t_gather(k + 2, 0)

      f_wait_gather(1)
      t_wait_store(k)
      compute(t0, f1, pidxt[k + 1], TAIL)
      t_store(k + 1)

      @pl.when(k + 3 < BPT)
      def _():
        t_gather(k + 3, 1)

      return carry

    lax.fori_loop(0, BPT // 2, t_ring, 0)

    t_wait_store(BPT - 1)

  return emb_kernel


_emb_kernel = _make_kernel()


@jax.jit
def kernel(input_ids, position_ids, token_table, position_table):
  ids = input_ids.astype(jnp.int32)
  pids = position_ids.astype(jnp.int32)
  idsf = ids[:, :FPB * CH].reshape(NT, FCP, CH)
  pidsf = pids[:, :FPB * CH].reshape(NT, FCP, CH)
  idst = jnp.pad(ids[:, FPB * CH:],
                 ((0, 0), (0, LANES - TAIL))).reshape(NT, BPT, LANES)
  pidst = jnp.pad(pids[:, FPB * CH:],
                  ((0, 0), (0, LANES - TAIL))).reshape(NT, BPT, LANES)
  return _emb_kernel(idsf, pidsf, idst, pidst, token_table, position_table)


# speedup vs baseline: 1.5098x; 1.0251x over previous
import functools

import jax
import jax.numpy as jnp
from jax import lax
from jax.experimental import pallas as pl
from jax.experimental.pallas import tpu as pltpu
from jax.experimental.pallas import tpu_sc as plsc

MAX_POS = 77
DIM = 768
HALF = DIM // 2
BATCH = 1024
SEQ = 77
CH = 16
FPB = 64 // CH
TAIL = SEQ - FPB * CH
NT = 16
BPT = BATCH // NT
FCP = BPT * FPB
LANES = 16


def _make_kernel():
  mesh = plsc.VectorSubcoreMesh(core_axis_name="c", subcore_axis_name="s")

  @functools.partial(
      pl.kernel,
      mesh=mesh,
      out_type=jax.ShapeDtypeStruct((BATCH, SEQ, DIM), jnp.float32),
      scratch_types=[
          pltpu.VMEM((FCP, CH), jnp.int32),
          pltpu.VMEM((FCP, CH), jnp.int32),
          pltpu.VMEM((BPT, LANES), jnp.int32),
          pltpu.VMEM((BPT, LANES), jnp.int32),
          pltpu.VMEM((MAX_POS, HALF), jnp.float32),
          pltpu.VMEM((CH, HALF), jnp.float32),
          pltpu.VMEM((CH, HALF), jnp.float32),
          pltpu.VMEM((TAIL, HALF), jnp.float32),
      ] + [pltpu.SemaphoreType.DMA for _ in range(5)],
  )
  def emb_kernel(idsf_hbm, pidsf_hbm, idst_hbm, pidst_hbm, tok_hbm, pos_hbm,
                 out_hbm, idxf, pidxf, idxt, pidxt, ptab,
                 f0, f1, t0, gf0, gf1, sf0, sf1, st0):
    fbufs = [f0, f1]
    fgsems = [gf0, gf1]
    fssems = [sf0, sf1]
    tile = lax.axis_index("s")
    c0 = lax.axis_index("c") * HALF

    def compute(dst, srcbuf, pvec, nrows):
      for i in range(nrows):
        p = pvec[i]

        @plsc.parallel_loop(0, HALF // LANES, step=8)
        def _(j, i=i, p=p, dst=dst, srcbuf=srcbuf):
          sls = [pl.ds((j + j8) * LANES, LANES) for j8 in range(8)]
          vals = [srcbuf[i, sl] + ptab[p, sl] for sl in sls]
          for j8 in range(8):
            dst[i, sls[j8]] = vals[j8]

    def f_gather(g, b):
      pltpu.async_copy(tok_hbm.at[idxf.at[g], pl.ds(c0, HALF)],
                       fbufs[b], fgsems[b])

    def f_wait_gather(b):
      pltpu.make_async_copy(tok_hbm.at[idxf.at[0], pl.ds(c0, HALF)],
                            fbufs[b], fgsems[b]).wait()

    def f_store_dst(g):
      k = g // FPB
      m = g - k * FPB
      return out_hbm.at[tile * BPT + k, pl.ds(m * CH, CH), pl.ds(c0, HALF)]

    def f_store(g, b):
      pltpu.async_copy(fbufs[b], f_store_dst(g), fssems[b])

    def f_wait_store(g, b):
      pltpu.make_async_copy(fbufs[b], f_store_dst(g), fssems[b]).wait()

    def t_gather(k, b):
      pltpu.async_copy(tok_hbm.at[idxt.at[k], pl.ds(c0, HALF)],
                       fbufs[b], fgsems[b])

    def t_store_dst(k):
      return out_hbm.at[tile * BPT + k, pl.ds(FPB * CH, TAIL),
                        pl.ds(c0, HALF)]

    def t_store(k):
      pltpu.async_copy(t0, t_store_dst(k), st0)

    def t_wait_store(k):
      pltpu.make_async_copy(t0, t_store_dst(k), st0).wait()

    pltpu.sync_copy(pos_hbm.at[:, pl.ds(c0, HALF)], ptab)
    pltpu.sync_copy(idsf_hbm.at[tile], idxf)
    pltpu.sync_copy(pidsf_hbm.at[tile], pidxf)
    pltpu.sync_copy(idst_hbm.at[tile], idxt)
    pltpu.sync_copy(pidst_hbm.at[tile], pidxt)

    f_gather(0, 0)
    f_gather(1, 1)

    def f_ring(i, carry):
      g = 2 * i
      f_wait_gather(0)
      compute(fbufs[0], fbufs[0], pidxf[g], CH)
      f_store(g, 0)
      f_wait_gather(1)
      compute(fbufs[1], fbufs[1], pidxf[g + 1], CH)
      f_store(g + 1, 1)

      @pl.when(i < FCP // 2 - 1)
      def _():
        f_wait_store(g, 0)
        f_gather(g + 2, 0)
        f_wait_store(g + 1, 1)
        f_gather(g + 3, 1)

      return carry

    lax.fori_loop(0, FCP // 2, f_ring, 0)

    f_wait_store(FCP - 2, 0)
    f_wait_store(FCP - 1, 1)

    t_gather(0, 0)
    t_gather(1, 1)

    def t_ring(kk, carry):
      k = 2 * kk
      f_wait_gather(0)

      @pl.when(kk > 0)
      def _():
        t_wait_store(k - 1)

      compute(t0, f0, pidxt[k], TAIL)
      t_store(k)

      @pl.when(k + 2 < BPT)
      def _():
        t_gather(k + 2, 0)

      f_wait_gather(1)
      t_wait_store(k)
      compute(t0, f1, pidxt[k + 1], TAIL)
      t_store(k + 1)

      @pl.when(k + 3 < BPT)
      def _():
        t_gather(k + 3, 1)

      return carry

    lax.fori_loop(0, BPT // 2, t_ring, 0)

    t_wait_store(BPT - 1)

  return emb_kernel


_emb_kernel = _make_kernel()


@jax.jit
def kernel(input_ids, position_ids, token_table, position_table):
  ids = input_ids.astype(jnp.int32)
  pids = position_ids.astype(jnp.int32)
  idsf = ids[:, :FPB * CH].reshape(NT, FCP, CH)
  pidsf = pids[:, :FPB * CH].reshape(NT, FCP, CH)
  idst = jnp.pad(ids[:, FPB * CH:],
                 ((0, 0), (0, LANES - TAIL))).reshape(NT, BPT, LANES)
  pidst = jnp.pad(pids[:, FPB * CH:],
                  ((0, 0), (0, LANES - TAIL))).reshape(NT, BPT, LANES)
  return _emb_kernel(idsf, pidsf, idst, pidst, token_table, position_table)
